# Initial kernel scaffold; baseline (speedup 1.0000x reference)
import jax
import jax.numpy as jnp
from jax.experimental import pallas as pl


def kernel(z_indices, random_ratios, rand_score, emb, W, b):
    raise NotImplementedError("write your pallas kernel here")


# speedup vs baseline: 3.1573x; 3.1573x over previous
import functools

import jax
import jax.numpy as jnp
from jax import lax
from jax.experimental import pallas as pl
from jax.experimental.pallas import tpu as pltpu

_B, _T = 64, 1024
_V, _D = 1024, 64
_MASK_ID = _V
_VP = 1152
_TILE = 512


def _mask_body(num_mask_ref, score_ref, z_ref, out_ref):
    u = lax.bitcast_convert_type(score_ref[...], jnp.int32)
    k = num_mask_ref[...]
    lo = jnp.zeros((_B, 1), jnp.int32)
    hi = jnp.full((_B, 1), jnp.int32(2**31 - 1))
    for _ in range(31):
        mid = lo + ((hi - lo) >> 1)
        cnt = jnp.sum((u <= mid).astype(jnp.int32), axis=1, keepdims=True)
        ge = cnt >= k
        hi = jnp.where(ge, mid, hi)
        lo = jnp.where(ge, lo, mid + 1)
    vstar = lo
    n_lt = jnp.sum((u < vstar).astype(jnp.int32), axis=1, keepdims=True)
    eq = u == vstar
    i0 = lax.broadcasted_iota(jnp.int32, (_T, _T), 0)
    i1 = lax.broadcasted_iota(jnp.int32, (_T, _T), 1)
    tri = (i0 < i1).astype(jnp.float32)
    pref = jnp.dot(eq.astype(jnp.float32), tri,
                   preferred_element_type=jnp.float32)
    quota = (k - n_lt).astype(jnp.float32)
    masked = (u < vstar) | (eq & (pref < quota))
    out_ref[...] = jnp.where(masked, _MASK_ID, z_ref[...])


def _head_body(idx_ref, emb_ref, w_ref, b_ref, out_ref):
    idx = idx_ref[...]
    oh = (idx == lax.broadcasted_iota(jnp.int32, (_TILE, _VP), 1)
          ).astype(jnp.float32)
    h = jnp.dot(oh, emb_ref[...], preferred_element_type=jnp.float32)
    out_ref[...] = (jnp.dot(h, w_ref[...], preferred_element_type=jnp.float32)
                    + b_ref[...])


def kernel(z_indices, random_ratios, rand_score, emb, W, b):
    num_mask = (jnp.cos(random_ratios * (jnp.pi / 2.0)) * _T).astype(
        jnp.int32).reshape(_B, 1)

    idx = pl.pallas_call(
        _mask_body,
        out_shape=jax.ShapeDtypeStruct((_B, _T), jnp.int32),
    )(num_mask, rand_score, z_indices.astype(jnp.int32))

    emb_pad = jnp.concatenate(
        [emb, jnp.zeros((_VP - (_V + 1), _D), jnp.float32)], axis=0)
    idx_col = idx.reshape(_B * _T, 1)

    n_tiles = (_B * _T) // _TILE
    logits = pl.pallas_call(
        _head_body,
        grid=(n_tiles,),
        in_specs=[
            pl.BlockSpec((_TILE, 1), lambda i: (i, 0)),
            pl.BlockSpec((_VP, _D), lambda i: (0, 0)),
            pl.BlockSpec((_D, _V), lambda i: (0, 0)),
            pl.BlockSpec((1, _V), lambda i: (0, 0)),
        ],
        out_specs=pl.BlockSpec((_TILE, _V), lambda i: (i, 0)),
        out_shape=jax.ShapeDtypeStruct((_B * _T, _V), jnp.float32),
    )(idx_col, emb_pad, W, b.reshape(1, _V))

    return (logits.reshape(_B, _T, _V), z_indices)
